# baseline (speedup 1.0000x reference)
import functools

import jax
import jax.numpy as jnp
from jax import lax
from jax.experimental import pallas as pl
from jax.experimental.pallas import tpu as pltpu
from jax.experimental.pallas import tpu_sc as plsc

_ROW_BLK = 256


def _matvec_body(x_ref, w_ref, b_ref, y_ref):
    wt = w_ref[...].reshape(1, -1)
    yb = lax.dot_general(
        wt, x_ref[...],
        dimension_numbers=(((1,), (1,)), ((), ())),
        preferred_element_type=jnp.float32,
    )
    y_ref[...] = (yb + b_ref[...]).reshape(_ROW_BLK)


def _matvec(x, W, b, n_pad):
    n, f = x.shape
    nblk = n_pad // _ROW_BLK
    return pl.pallas_call(
        _matvec_body,
        grid=(nblk,),
        in_specs=[
            pl.BlockSpec((_ROW_BLK, f), lambda i: (i, 0)),
            pl.BlockSpec((f, 1), lambda i: (0, 0)),
            pl.BlockSpec((1, 1), lambda i: (0, 0)),
        ],
        out_specs=pl.BlockSpec((_ROW_BLK,), lambda i: (i,)),
        out_shape=jax.ShapeDtypeStruct((n_pad,), jnp.float32),
    )(x, W, b.reshape(1, 1))


@functools.partial(jax.jit, static_argnames=("n_pad", "n_seg"))
def _segsum_call(y, seg_ids, n_pad, n_seg):
    NW = 16
    L = 16
    chunk = n_pad // NW
    nvec = chunk // L
    acc_len = ((n_seg + 1 + L - 1) // L) * L
    ncols = n_seg // L
    reps = (ncols + NW - 1) // NW

    mesh = plsc.VectorSubcoreMesh(
        core_axis_name="c", subcore_axis_name="s", num_cores=1)

    @functools.partial(
        pl.kernel,
        mesh=mesh,
        out_type=jax.ShapeDtypeStruct((n_seg,), jnp.float32),
        compiler_params=pltpu.CompilerParams(needs_layout_passes=False),
        scratch_types=[
            pltpu.VMEM((chunk,), jnp.float32),
            pltpu.VMEM((chunk,), jnp.int32),
            pltpu.VMEM((acc_len,), jnp.float32),
            pltpu.VMEM((NW * L,), jnp.float32),
            pltpu.VMEM_SHARED((acc_len * NW,), jnp.float32),
        ],
    )
    def segsum(y_hbm, idx_hbm, out_hbm, y_v, idx_v, acc, colbuf, shared):
        wid = lax.axis_index("s")
        base = wid * chunk
        pltpu.sync_copy(y_hbm.at[pl.ds(base, chunk)], y_v)
        pltpu.sync_copy(idx_hbm.at[pl.ds(base, chunk)], idx_v)
        zeros = jnp.zeros((L,), jnp.float32)
        for j in range(acc_len // L):
            acc[pl.ds(j * L, L)] = zeros

        def body(i, carry):
            off = pl.multiple_of(i * L, L)
            iv = idx_v[pl.ds(off, L)]
            yv = y_v[pl.ds(off, L)]
            plsc.addupdate_scatter(acc, [iv], yv)
            return carry

        lax.fori_loop(0, nvec, body, 0)

        for j in range(acc_len // L):
            pltpu.sync_copy(acc.at[pl.ds(j * L, L)],
                            shared.at[pl.ds(j * NW * L + wid * L, L)])
        plsc.subcore_barrier()

        for rep in range(reps):
            col = wid + rep * NW

            @pl.when(col < ncols)
            def _():
                pltpu.sync_copy(shared.at[pl.ds(col * NW * L, NW * L)],
                                colbuf)
                tot = zeros
                for k in range(NW):
                    tot = tot + colbuf[pl.ds(k * L, L)]
                acc[pl.ds(0, L)] = tot
                pltpu.sync_copy(acc.at[pl.ds(0, L)],
                                out_hbm.at[pl.ds(col * L, L)])

    return segsum(y, seg_ids)


def kernel(x, atomic_subsystem_counts, W, b):
    n, _ = x.shape
    n_seg = atomic_subsystem_counts.shape[0]
    counts = atomic_subsystem_counts.astype(jnp.int32)
    seg_ids = jnp.repeat(
        jnp.arange(n_seg, dtype=jnp.int32), counts, total_repeat_length=n)
    n_pad = ((n + _ROW_BLK - 1) // _ROW_BLK) * _ROW_BLK
    seg_ids_pad = jnp.concatenate(
        [seg_ids, jnp.full((n_pad - n,), n_seg, jnp.int32)])
    y = _matvec(x, W, b, n_pad)
    out = _segsum_call(y, seg_ids_pad, n_pad=n_pad, n_seg=n_seg)
    return out.reshape(n_seg, 1)


# speedup vs baseline: 1.2649x; 1.2649x over previous
import functools

import jax
import jax.numpy as jnp
from jax import lax
from jax.experimental import pallas as pl
from jax.experimental.pallas import tpu as pltpu
from jax.experimental.pallas import tpu_sc as plsc

_ROW_BLK = 2048


def _matvec_body(x_ref, w_ref, b_ref, y_ref):
    wt = w_ref[...].reshape(1, -1)
    yb = lax.dot_general(
        wt, x_ref[...],
        dimension_numbers=(((1,), (1,)), ((), ())),
        preferred_element_type=jnp.float32,
    )
    y_ref[...] = (yb + b_ref[...]).reshape(_ROW_BLK)


def _matvec(x, W, b, n_pad):
    n, f = x.shape
    nblk = n_pad // _ROW_BLK
    return pl.pallas_call(
        _matvec_body,
        grid=(nblk,),
        in_specs=[
            pl.BlockSpec((_ROW_BLK, f), lambda i: (i, 0)),
            pl.BlockSpec((f, 1), lambda i: (0, 0)),
            pl.BlockSpec((1, 1), lambda i: (0, 0)),
        ],
        out_specs=pl.BlockSpec((_ROW_BLK,), lambda i: (i,)),
        out_shape=jax.ShapeDtypeStruct((n_pad,), jnp.float32),
    )(x, W, b.reshape(1, 1))


@functools.partial(jax.jit, static_argnames=("n_pad", "n_seg"))
def _segsum_call(y, seg_ids, n_pad, n_seg):
    NW = 16
    L = 16
    chunk = n_pad // NW
    nvec = chunk // L
    acc_len = ((n_seg + 1 + L - 1) // L) * L
    ncols = n_seg // L
    reps = (ncols + NW - 1) // NW

    mesh = plsc.VectorSubcoreMesh(
        core_axis_name="c", subcore_axis_name="s", num_cores=1)

    @functools.partial(
        pl.kernel,
        mesh=mesh,
        out_type=jax.ShapeDtypeStruct((n_seg,), jnp.float32),
        compiler_params=pltpu.CompilerParams(needs_layout_passes=False),
        scratch_types=[
            pltpu.VMEM((chunk,), jnp.float32),
            pltpu.VMEM((chunk,), jnp.int32),
            pltpu.VMEM((acc_len,), jnp.float32),
            pltpu.VMEM((NW * L,), jnp.float32),
            pltpu.VMEM_SHARED((acc_len * NW,), jnp.float32),
        ],
    )
    def segsum(y_hbm, idx_hbm, out_hbm, y_v, idx_v, acc, colbuf, shared):
        wid = lax.axis_index("s")
        base = wid * chunk
        pltpu.sync_copy(y_hbm.at[pl.ds(base, chunk)], y_v)
        pltpu.sync_copy(idx_hbm.at[pl.ds(base, chunk)], idx_v)
        zeros = jnp.zeros((L,), jnp.float32)
        for j in range(acc_len // L):
            acc[pl.ds(j * L, L)] = zeros

        def body(i, carry):
            off = pl.multiple_of(i * L, L)
            iv = idx_v[pl.ds(off, L)]
            yv = y_v[pl.ds(off, L)]
            plsc.addupdate_scatter(acc, [iv], yv)
            return carry

        lax.fori_loop(0, nvec, body, 0)

        for j in range(acc_len // L):
            pltpu.sync_copy(acc.at[pl.ds(j * L, L)],
                            shared.at[pl.ds(j * NW * L + wid * L, L)])
        plsc.subcore_barrier()

        for rep in range(reps):
            col = wid + rep * NW

            @pl.when(col < ncols)
            def _():
                pltpu.sync_copy(shared.at[pl.ds(col * NW * L, NW * L)],
                                colbuf)
                tot = zeros
                for k in range(NW):
                    tot = tot + colbuf[pl.ds(k * L, L)]
                acc[pl.ds(0, L)] = tot
                pltpu.sync_copy(acc.at[pl.ds(0, L)],
                                out_hbm.at[pl.ds(col * L, L)])

    return segsum(y, seg_ids)


def kernel(x, atomic_subsystem_counts, W, b):
    n, _ = x.shape
    n_seg = atomic_subsystem_counts.shape[0]
    counts = atomic_subsystem_counts.astype(jnp.int32)
    seg_ids = jnp.repeat(
        jnp.arange(n_seg, dtype=jnp.int32), counts, total_repeat_length=n)
    n_pad = ((n + _ROW_BLK - 1) // _ROW_BLK) * _ROW_BLK
    seg_ids_pad = jnp.concatenate(
        [seg_ids, jnp.full((n_pad - n,), n_seg, jnp.int32)])
    y = _matvec(x, W, b, n_pad)
    out = _segsum_call(y, seg_ids_pad, n_pad=n_pad, n_seg=n_seg)
    return out.reshape(n_seg, 1)


# speedup vs baseline: 1.2868x; 1.0174x over previous
import functools

import jax
import jax.numpy as jnp
from jax import lax
from jax.experimental import pallas as pl
from jax.experimental.pallas import tpu as pltpu
from jax.experimental.pallas import tpu_sc as plsc

_ROW_BLK = 7168


def _matvec_body(x_ref, w_ref, b_ref, y_ref):
    wt = w_ref[...].reshape(1, -1)
    yb = lax.dot_general(
        wt, x_ref[...],
        dimension_numbers=(((1,), (1,)), ((), ())),
        preferred_element_type=jnp.float32,
    )
    y_ref[...] = (yb + b_ref[...]).reshape(_ROW_BLK)


def _matvec(x, W, b, n_pad):
    n, f = x.shape
    nblk = n_pad // _ROW_BLK
    return pl.pallas_call(
        _matvec_body,
        grid=(nblk,),
        in_specs=[
            pl.BlockSpec((_ROW_BLK, f), lambda i: (i, 0)),
            pl.BlockSpec((f, 1), lambda i: (0, 0)),
            pl.BlockSpec((1, 1), lambda i: (0, 0)),
        ],
        out_specs=pl.BlockSpec((_ROW_BLK,), lambda i: (i,)),
        out_shape=jax.ShapeDtypeStruct((n_pad,), jnp.float32),
    )(x, W, b.reshape(1, 1))


@functools.partial(jax.jit, static_argnames=("n_pad", "n_seg"))
def _segsum_call(y, seg_ids, n_pad, n_seg):
    NW = 16
    L = 16
    chunk = n_pad // NW
    nvec = chunk // L
    acc_len = ((n_seg + 1 + L - 1) // L) * L
    ncols = n_seg // L
    reps = (ncols + NW - 1) // NW

    mesh = plsc.VectorSubcoreMesh(
        core_axis_name="c", subcore_axis_name="s", num_cores=1)

    @functools.partial(
        pl.kernel,
        mesh=mesh,
        out_type=jax.ShapeDtypeStruct((n_seg,), jnp.float32),
        compiler_params=pltpu.CompilerParams(needs_layout_passes=False),
        scratch_types=[
            pltpu.VMEM((chunk,), jnp.float32),
            pltpu.VMEM((chunk,), jnp.int32),
            pltpu.VMEM((acc_len,), jnp.float32),
            pltpu.VMEM((NW * L,), jnp.float32),
            pltpu.VMEM_SHARED((acc_len * NW,), jnp.float32),
        ],
    )
    def segsum(y_hbm, idx_hbm, out_hbm, y_v, idx_v, acc, colbuf, shared):
        wid = lax.axis_index("s")
        base = wid * chunk
        pltpu.sync_copy(y_hbm.at[pl.ds(base, chunk)], y_v)
        pltpu.sync_copy(idx_hbm.at[pl.ds(base, chunk)], idx_v)
        zeros = jnp.zeros((L,), jnp.float32)
        for j in range(acc_len // L):
            acc[pl.ds(j * L, L)] = zeros

        def body(i, carry):
            off = pl.multiple_of(i * L, L)
            iv = idx_v[pl.ds(off, L)]
            yv = y_v[pl.ds(off, L)]
            plsc.addupdate_scatter(acc, [iv], yv)
            return carry

        lax.fori_loop(0, nvec, body, 0)

        for j in range(acc_len // L):
            pltpu.sync_copy(acc.at[pl.ds(j * L, L)],
                            shared.at[pl.ds(j * NW * L + wid * L, L)])
        plsc.subcore_barrier()

        for rep in range(reps):
            col = wid + rep * NW

            @pl.when(col < ncols)
            def _():
                pltpu.sync_copy(shared.at[pl.ds(col * NW * L, NW * L)],
                                colbuf)
                tot = zeros
                for k in range(NW):
                    tot = tot + colbuf[pl.ds(k * L, L)]
                acc[pl.ds(0, L)] = tot
                pltpu.sync_copy(acc.at[pl.ds(0, L)],
                                out_hbm.at[pl.ds(col * L, L)])

    return segsum(y, seg_ids)


def kernel(x, atomic_subsystem_counts, W, b):
    n, _ = x.shape
    n_seg = atomic_subsystem_counts.shape[0]
    counts = atomic_subsystem_counts.astype(jnp.int32)
    seg_ids = jnp.repeat(
        jnp.arange(n_seg, dtype=jnp.int32), counts, total_repeat_length=n)
    n_pad = ((n + _ROW_BLK - 1) // _ROW_BLK) * _ROW_BLK
    seg_ids_pad = jnp.concatenate(
        [seg_ids, jnp.full((n_pad - n,), n_seg, jnp.int32)])
    y = _matvec(x, W, b, n_pad)
    out = _segsum_call(y, seg_ids_pad, n_pad=n_pad, n_seg=n_seg)
    return out.reshape(n_seg, 1)
